# baseline (speedup 1.0000x reference)
import math

import jax
import jax.numpy as jnp
from jax.experimental import pallas as pl

N_NODES, N_EDGES, F_IN, HID, N_CLS = 10000, 320000, 128, 50, 4
RATIO = 0.5


def _gcn_conv(x, src, dst, ew, W, b):
    N = x.shape[0]
    ar = jnp.arange(N, dtype=src.dtype)
    srcl = jnp.concatenate([src, ar])
    dstl = jnp.concatenate([dst, ar])
    ewl = jnp.concatenate([ew, jnp.ones((N,), x.dtype)])
    deg = jnp.zeros((N,), x.dtype).at[dstl].add(ewl)
    dinv = jnp.where(deg > 0, deg ** -0.5, 0.0)
    norm = dinv[srcl] * dinv[dstl] * ewl
    xw = x @ W
    out = jnp.zeros((N, W.shape[1]), x.dtype).at[dstl].add(xw[srcl] * norm[:, None])
    return out + b


def _topk_pool(x, src, dst, ew, batch, weight, ratio):
    N = x.shape[0]
    score = jnp.tanh((x @ weight) / jnp.linalg.norm(weight))
    k = int(math.ceil(ratio * N))
    topv, perm = jax.lax.top_k(score, k)
    x_new = x[perm] * topv[:, None]
    idx = jnp.full((N,), -1, dtype=src.dtype).at[perm].set(jnp.arange(k, dtype=src.dtype))
    ns = idx[src]
    nd = idx[dst]
    valid = (ns >= 0) & (nd >= 0)
    ew_new = ew * valid.astype(x.dtype)
    ns = jnp.where(valid, ns, 0)
    nd = jnp.where(valid, nd, 0)
    batch_new = batch[perm]
    return x_new, ns, nd, ew_new, batch_new


def _final_kernel(g_ref, wfc_ref, bfc_ref, out_ref):
    logits = jnp.dot(g_ref[...], wfc_ref[...], preferred_element_type=jnp.float32)
    logits = logits + bfc_ref[...]
    col = jax.lax.broadcasted_iota(jnp.int32, (8, 128), 1)
    neg = jnp.where(col < N_CLS, logits, -jnp.inf)
    m = jnp.max(neg, axis=1, keepdims=True)
    e = jnp.where(col < N_CLS, jnp.exp(logits - m), 0.0)
    lse = jnp.log(jnp.sum(e, axis=1, keepdims=True)) + m
    out_ref[...] = logits - lse


def kernel(x, edge_index, batch, W1, b1, W2, b2, p1, p2, Wfc, bfc):
    src, dst = edge_index[0], edge_index[1]
    ew = jnp.ones((src.shape[0],), x.dtype)
    x1 = _gcn_conv(x, src, dst, ew, W1, b1)
    x1p, s1, d1, ew1, bt1 = _topk_pool(x1, src, dst, ew, batch, p1, RATIO)
    x2 = _gcn_conv(x1p, s1, d1, ew1, W2, b2)
    g1 = jnp.concatenate([jnp.max(x1p, axis=0), jnp.mean(x1p, axis=0)])
    x2p, s2, d2, ew2, bt2 = _topk_pool(x2, s1, d1, ew1, bt1, p2, RATIO)
    g2 = jnp.concatenate([jnp.max(x2p, axis=0), jnp.mean(x2p, axis=0)])
    xx = jnp.concatenate([g1, g2])
    g_pad = jnp.zeros((8, 256), jnp.float32).at[0, :200].set(xx)
    wfc_pad = jnp.zeros((256, 128), jnp.float32).at[:200, :N_CLS].set(Wfc)
    bfc_pad = jnp.zeros((8, 128), jnp.float32).at[:, :N_CLS].set(bfc)
    out = pl.pallas_call(
        _final_kernel,
        out_shape=jax.ShapeDtypeStruct((8, 128), jnp.float32),
    )(g_pad, wfc_pad, bfc_pad)
    return out[0:1, 0:N_CLS]


# speedup vs baseline: 26.5793x; 26.5793x over previous
import functools
import math

import jax
import jax.numpy as jnp
from jax import lax
from jax.experimental import pallas as pl
from jax.experimental.pallas import tpu as pltpu
from jax.experimental.pallas import tpu_sc as plsc

N = 10000
E = 320000
F_IN = 128
HID = 50
HP = 64
WH = 16
NCLS = 4
K1 = int(math.ceil(0.5 * N))
K2 = int(math.ceil(0.5 * K1))
NPAD = 10240
NROW2D = NPAD // 128

NC = 2
NS = 16
NW = NC * NS
CH = 128
NCHUNK = E // CH
RPT = 632
ACCN = RPT * NS

GB = 1000
NB = N // GB

_SC_MESH = plsc.VectorSubcoreMesh(core_axis_name="c", subcore_axis_name="s")


def _sc_gather_scatter_add(width):
    @functools.partial(
        pl.kernel,
        out_type=jax.ShapeDtypeStruct((NC, ACCN, width), jnp.float32),
        mesh=_SC_MESH,
        compiler_params=pltpu.CompilerParams(use_tc_tiling_on_sc=False),
        scratch_types=[
            pltpu.VMEM((CH,), jnp.int32),
            pltpu.VMEM((CH,), jnp.int32),
            pltpu.VMEM((CH, width), jnp.float32),
            pltpu.VMEM_SHARED((ACCN, width), jnp.float32),
            pltpu.SemaphoreType.DMA,
        ],
    )
    def k(table, srcv, dstv, zrows, out, sidx, didx, rows, acc, sem):
        c = lax.axis_index("c")
        s = lax.axis_index("s")
        wid = s * NC + c
        pltpu.sync_copy(zrows, acc.at[pl.ds(s * RPT, RPT)])
        plsc.subcore_barrier()
        nch = (NCHUNK - wid + NW - 1) // NW

        def step(i, carry):
            off = pl.multiple_of((wid + i * NW) * CH, CH)
            pltpu.sync_copy(srcv.at[pl.ds(off, CH)], sidx)
            pltpu.sync_copy(dstv.at[pl.ds(off, CH)], didx)
            pltpu.async_copy(table.at[sidx], rows, sem).wait()
            pltpu.sync_copy(rows, acc.at[didx], add=True)
            return carry

        lax.fori_loop(0, nch, step, 0)
        plsc.subcore_barrier()
        pltpu.sync_copy(acc.at[pl.ds(s * RPT, RPT)],
                        out.at[c, pl.ds(s * RPT, RPT)])

    return k


def _sc_scatter_ones():
    @functools.partial(
        pl.kernel,
        out_type=jax.ShapeDtypeStruct((NC, ACCN, WH), jnp.float32),
        mesh=_SC_MESH,
        compiler_params=pltpu.CompilerParams(use_tc_tiling_on_sc=False),
        scratch_types=[
            pltpu.VMEM((CH,), jnp.int32),
            pltpu.VMEM((CH, WH), jnp.float32),
            pltpu.VMEM_SHARED((ACCN, WH), jnp.float32),
        ],
    )
    def k(dstv, onesr, zrows, out, didx, rows, acc):
        c = lax.axis_index("c")
        s = lax.axis_index("s")
        wid = s * NC + c
        pltpu.sync_copy(zrows, acc.at[pl.ds(s * RPT, RPT)])
        pltpu.sync_copy(onesr, rows)
        plsc.subcore_barrier()
        nch = (NCHUNK - wid + NW - 1) // NW

        def step(i, carry):
            off = pl.multiple_of((wid + i * NW) * CH, CH)
            pltpu.sync_copy(dstv.at[pl.ds(off, CH)], didx)
            pltpu.sync_copy(rows, acc.at[didx], add=True)
            return carry

        lax.fori_loop(0, nch, step, 0)
        plsc.subcore_barrier()
        pltpu.sync_copy(acc.at[pl.ds(s * RPT, RPT)],
                        out.at[c, pl.ds(s * RPT, RPT)])

    return k


def _k12_body(x_ref, w_ref, hist_ref, xw_ref, xs_ref):
    xw = jnp.dot(x_ref[...], w_ref[...], preferred_element_type=jnp.float32)
    d = hist_ref[0] + hist_ref[1]
    dinv = lax.rsqrt(1.0 + d[:, 0:1])
    xw_ref[...] = xw
    xs_ref[...] = xw * dinv


def _k3a_body(a_ref, xw_ref, hist_ref, b_ref, p_ref, x1_ref, sc_ref):
    d = hist_ref[0] + hist_ref[1]
    dinv = lax.rsqrt(1.0 + d[:, 0:1])
    agg = a_ref[0] + a_ref[1]
    xw = xw_ref[...]
    x1 = dinv * agg + (dinv * dinv) * xw + b_ref[...]
    x1_ref[...] = x1
    pn = p_ref[...]
    pnorm2 = jnp.sum(pn[:, 0:1] * pn[:, 0:1])
    sc = jnp.dot(x1, pn, preferred_element_type=jnp.float32) * lax.rsqrt(pnorm2)
    sc_ref[...] = jnp.tanh(sc)


def _select_body(k_const, sc_ref, msk_ref, m_ref, v_ref):
    sc = sc_ref[...]
    ridx = lax.broadcasted_iota(jnp.int32, (NROW2D, 128), 0)
    cidx = lax.broadcasted_iota(jnp.int32, (NROW2D, 128), 1)
    idx = ridx * 128 + cidx
    valid = (idx < N) & (msk_ref[...] > 0)
    bits = lax.bitcast_convert_type(sc, jnp.uint32)
    key = jnp.where((bits >> 31) == 0, bits | jnp.uint32(0x80000000), ~bits)
    key = jnp.where(valid, key, jnp.uint32(0))

    def tstep(b, t):
        sh = lax.shift_left(jnp.uint32(1), jnp.uint32(31) - b.astype(jnp.uint32))
        t_try = t | sh
        c = jnp.sum((key >= t_try).astype(jnp.int32))
        return jnp.where(c >= k_const, t_try, t)

    tthr = lax.fori_loop(0, 32, tstep, jnp.uint32(0))
    gt = key > tthr
    c_gt = jnp.sum(gt.astype(jnp.int32))
    r = k_const - c_gt
    eq = key == tthr

    def jstep(b, j):
        j_try = j | lax.shift_left(jnp.int32(1), jnp.int32(14) - b)
        c = jnp.sum((eq & (idx < j_try)).astype(jnp.int32))
        return jnp.where(c <= r, j_try, j)

    jcut = lax.fori_loop(0, 15, jstep, jnp.int32(0))
    sel = gt | (eq & (idx < jcut))
    m_ref[...] = sel.astype(jnp.float32)
    v_ref[...] = jnp.where(sel, sc, 0.0)


def _k3c_body(x1_ref, sel_ref, m_ref, w2_ref, xw2_ref, g_ref):
    i = pl.program_id(0)
    y = x1_ref[...] * sel_ref[...][:, 0:1]
    xw2_ref[...] = jnp.dot(y, w2_ref[...], preferred_element_type=jnp.float32)
    mcol = m_ref[...][:, 0:1] > 0
    bmax = jnp.max(jnp.where(mcol, y, -jnp.inf), axis=0, keepdims=True)
    bsum = jnp.sum(jnp.where(mcol, y, 0.0), axis=0, keepdims=True)

    @pl.when(i == 0)
    def _():
        g_ref[...] = jnp.concatenate([bmax, bsum], axis=0)

    @pl.when(i > 0)
    def _():
        prev = g_ref[...]
        g_ref[...] = jnp.concatenate(
            [jnp.maximum(prev[0:1], bmax), prev[1:2] + bsum], axis=0)

    @pl.when(i == NB - 1)
    def _():
        g = g_ref[...]
        g_ref[...] = jnp.concatenate([g[0:1], g[1:2] * (1.0 / K1)], axis=0)


def _k4_body(xw2_ref, hist_ref, m_ref, xs2_ref, dinv_ref):
    d = hist_ref[0] + hist_ref[1]
    mcol = m_ref[...][:, 0:1] > 0
    dinv = jnp.where(mcol, lax.rsqrt(1.0 + d[:, 0:1]), 0.0)
    xs2_ref[...] = xw2_ref[...] * dinv
    dinv_ref[...] = jnp.broadcast_to(dinv, (GB, 8))


def _k5a_body(a_ref, xw2_ref, dinv_ref, b_ref, p_ref, x2_ref, sc_ref):
    dinv = dinv_ref[...][:, 0:1]
    agg = a_ref[0] + a_ref[1]
    xw2 = xw2_ref[...]
    x2 = dinv * agg + (dinv * dinv) * xw2 + b_ref[...]
    x2_ref[...] = x2
    pn = p_ref[...]
    pnorm2 = jnp.sum(pn[:, 0:1] * pn[:, 0:1])
    sc = jnp.dot(x2, pn, preferred_element_type=jnp.float32) * lax.rsqrt(pnorm2)
    sc_ref[...] = jnp.tanh(sc)


def _k5c_body(x2_ref, sel_ref, m_ref, g_ref):
    i = pl.program_id(0)
    y = x2_ref[...] * sel_ref[...][:, 0:1]
    mcol = m_ref[...][:, 0:1] > 0
    bmax = jnp.max(jnp.where(mcol, y, -jnp.inf), axis=0, keepdims=True)
    bsum = jnp.sum(jnp.where(mcol, y, 0.0), axis=0, keepdims=True)

    @pl.when(i == 0)
    def _():
        g_ref[...] = jnp.concatenate([bmax, bsum], axis=0)

    @pl.when(i > 0)
    def _():
        prev = g_ref[...]
        g_ref[...] = jnp.concatenate(
            [jnp.maximum(prev[0:1], bmax), prev[1:2] + bsum], axis=0)

    @pl.when(i == NB - 1)
    def _():
        g = g_ref[...]
        g_ref[...] = jnp.concatenate([g[0:1], g[1:2] * (1.0 / K2)], axis=0)


def _k6_body(g1_ref, g2_ref, wfc_ref, bfc_ref, out_ref):
    dot = functools.partial(jnp.dot, preferred_element_type=jnp.float32)
    logits = (dot(g1_ref[0:1], wfc_ref[0]) + dot(g1_ref[1:2], wfc_ref[1])
              + dot(g2_ref[0:1], wfc_ref[2]) + dot(g2_ref[1:2], wfc_ref[3]))
    logits = logits + bfc_ref[...]
    col = lax.broadcasted_iota(jnp.int32, (1, 128), 1)
    neg = jnp.where(col < NCLS, logits, -jnp.inf)
    m = jnp.max(neg)
    e = jnp.where(col < NCLS, jnp.exp(logits - m), 0.0)
    lse = jnp.log(jnp.sum(e)) + m
    out_ref[...] = jnp.broadcast_to(logits - lse, (8, 128))


def _rb(width):
    return pl.BlockSpec((GB, width), lambda i: (i, 0))


def _pb(shape):
    return pl.BlockSpec(shape, lambda i: tuple(0 for _ in shape))


def _hb(width):
    return pl.BlockSpec((NC, GB, width), lambda i: (0, i, 0))


def _f32(*shape):
    return jax.ShapeDtypeStruct(shape, jnp.float32)


def _pad2d(flat8):
    return jnp.pad(flat8[:, 0], (0, NPAD - N)).reshape(NROW2D, 128)


def _torep(arr2d):
    flat = arr2d.reshape(NPAD)[:N]
    return jnp.broadcast_to(flat[:, None], (N, 8))


def kernel(x, edge_index, batch, W1, b1, W2, b2, p1, p2, Wfc, bfc):
    f32 = jnp.float32
    src = edge_index[0].astype(jnp.int32)
    dst = edge_index[1].astype(jnp.int32)

    W1p = jnp.zeros((F_IN, HP), f32).at[:, :HID].set(W1)
    W2p = jnp.zeros((HP, HP), f32).at[:HID, :HID].set(W2)
    b1p = jnp.zeros((1, HP), f32).at[0, :HID].set(b1)
    b2p = jnp.zeros((1, HP), f32).at[0, :HID].set(b2)
    p1rep = jnp.broadcast_to(
        jnp.zeros((HP,), f32).at[:HID].set(p1)[:, None], (HP, 8))
    p2rep = jnp.broadcast_to(
        jnp.zeros((HP,), f32).at[:HID].set(p2)[:, None], (HP, 8))
    wfc_pad = jnp.zeros((4, HP, 128), f32)
    for blk in range(4):
        wfc_pad = wfc_pad.at[blk, :HID, :NCLS].set(Wfc[blk * HID:(blk + 1) * HID])
    bfc_pad = jnp.zeros((1, 128), f32).at[0, :NCLS].set(bfc)
    zrows_h = jnp.zeros((RPT, WH), f32)
    zrows_f = jnp.zeros((RPT, HP), f32)
    ones_r = jnp.ones((CH, WH), f32)
    ones2d = jnp.ones((NROW2D, 128), f32)

    hist1 = _sc_scatter_ones()(dst, ones_r, zrows_h)

    xw1, xs1 = pl.pallas_call(
        _k12_body,
        grid=(NB,),
        in_specs=[_rb(F_IN), _pb((F_IN, HP)), _hb(WH)],
        out_specs=[_rb(HP), _rb(HP)],
        out_shape=[_f32(N, HP), _f32(N, HP)],
    )(x, W1p, hist1)

    A1 = _sc_gather_scatter_add(HP)(xs1, src, dst, zrows_f)

    x1, sc1 = pl.pallas_call(
        _k3a_body,
        grid=(NB,),
        in_specs=[_hb(HP), _rb(HP), _hb(WH), _pb((1, HP)), _pb((HP, 8))],
        out_specs=[_rb(HP), _rb(8)],
        out_shape=[_f32(N, HP), _f32(N, 8)],
    )(A1, xw1, hist1, b1p, p1rep)

    m1_2d, sel1_2d = pl.pallas_call(
        functools.partial(_select_body, K1),
        out_shape=[_f32(NROW2D, 128), _f32(NROW2D, 128)],
    )(_pad2d(sc1), ones2d)
    m1rep = _torep(m1_2d)
    sel1rep = _torep(sel1_2d)

    xw2, g1 = pl.pallas_call(
        _k3c_body,
        grid=(NB,),
        in_specs=[_rb(HP), _rb(8), _rb(8), _pb((HP, HP))],
        out_specs=[_rb(HP), _pb((2, HP))],
        out_shape=[_f32(N, HP), _f32(2, HP)],
    )(x1, sel1rep, m1rep, W2p)

    t2 = jnp.broadcast_to(m1_2d.reshape(NPAD)[:N, None], (N, WH))
    hist2 = _sc_gather_scatter_add(WH)(t2, src, dst, zrows_h)

    xs2, dinv2rep = pl.pallas_call(
        _k4_body,
        grid=(NB,),
        in_specs=[_rb(HP), _hb(WH), _rb(8)],
        out_specs=[_rb(HP), _rb(8)],
        out_shape=[_f32(N, HP), _f32(N, 8)],
    )(xw2, hist2, m1rep)

    A2 = _sc_gather_scatter_add(HP)(xs2, src, dst, zrows_f)

    x2, sc2 = pl.pallas_call(
        _k5a_body,
        grid=(NB,),
        in_specs=[_hb(HP), _rb(HP), _rb(8), _pb((1, HP)), _pb((HP, 8))],
        out_specs=[_rb(HP), _rb(8)],
        out_shape=[_f32(N, HP), _f32(N, 8)],
    )(A2, xw2, dinv2rep, b2p, p2rep)

    m2_2d, sel2_2d = pl.pallas_call(
        functools.partial(_select_body, K2),
        out_shape=[_f32(NROW2D, 128), _f32(NROW2D, 128)],
    )(_pad2d(sc2), m1_2d)

    g2 = pl.pallas_call(
        _k5c_body,
        grid=(NB,),
        in_specs=[_rb(HP), _rb(8), _rb(8)],
        out_specs=_pb((2, HP)),
        out_shape=_f32(2, HP),
    )(x2, _torep(sel2_2d), _torep(m2_2d))

    out = pl.pallas_call(
        _k6_body,
        out_shape=_f32(8, 128),
    )(g1, g2, wfc_pad, bfc_pad)
    return out[0:1, 0:NCLS]
